# baseline (speedup 1.0000x reference)
import functools

import jax
import jax.numpy as jnp
from jax import lax
from jax.experimental import pallas as pl
from jax.experimental.pallas import tpu as pltpu
from jax.experimental.pallas import tpu_sc as plsc

N_NODES = 10000
N_EDGES = 320000
D = 128

NC = 2
NS = 16
CHUNK = 32
NCH = 327680 // (NS * CHUNK)
BLKCH = 32
E_PAD = NS * NCH * CHUNK
TBL_ROWS = 10240
HALF = 5008
STG = HALF // NS
ACC_ROWS = 10016
RPT = ACC_ROWS // NS
JUNK = N_NODES
CAP = 128
CNT_W = 16
BLK1 = 512
BLK = 400

_SC_PARAMS = pltpu.CompilerParams(use_tc_tiling_on_sc=False,
                                  needs_layout_passes=False)


def _agg_kernel(idx_hbm, table_hbm, zeros_hbm, out_hbm,
                sdblk, cpk, fs0, fd0, fs1, fd1, rows0, rows1, acc, tbl,
                semg0, sems0, semg1, sems1):
    c = lax.axis_index("c")
    s = lax.axis_index("s")
    base = c * HALF
    pltpu.sync_copy(zeros_hbm, acc.at[pl.ds(s * RPT, RPT)])
    pltpu.sync_copy(table_hbm.at[pl.ds(base + s * STG, STG)],
                    tbl.at[pl.ds(s * STG, STG)])
    plsc.subcore_barrier()

    def fire_on(k, fs_a, fd_a, rows_a, semg_a, sems_a,
                fs_b, fd_b, rows_b, semg_b, sems_b):
        @pl.when(k >= 1)
        def _():
            pltpu.make_async_copy(tbl.at[fs_b], rows_b, semg_b).wait()
            pltpu.async_copy(rows_b, acc.at[fd_b], sems_b, add=True)

        @pl.when(k >= 2)
        def _():
            pltpu.make_async_copy(rows_a, acc.at[fd_a], sems_a).wait()

        for g in range(CHUNK // 16):
            v = cpk[pl.ds(g * 16, 16)]
            fs_a[pl.ds(g * 16, 16)] = lax.shift_right_logical(v, 14)
            fd_a[pl.ds(g * 16, 16)] = lax.bitwise_and(v, 16383)
        pltpu.async_copy(tbl.at[fs_a], rows_a, semg_a)
        cpk[pl.ds(0, 16)] = cpk[pl.ds(CHUNK, 16)]

    def fire(k):
        @pl.when(k % 2 == 0)
        def _():
            fire_on(k, fs0, fd0, rows0, semg0, sems0,
                    fs1, fd1, rows1, semg1, sems1)

        @pl.when(k % 2 == 1)
        def _():
            fire_on(k, fs1, fd1, rows1, semg1, sems1,
                    fs0, fd0, rows0, semg0, sems0)

    def block_body(jb, carry):
        fill, k = carry
        pltpu.sync_copy(idx_hbm.at[s, pl.ds(jb * BLKCH, BLKCH)], sdblk)
        for jj in range(BLKCH):
            for g in range(CHUNK // 16):
                srcv = sdblk[jj, 0, pl.ds(g * 16, 16)]
                dstv = sdblk[jj, 1, pl.ds(g * 16, 16)]
                loc = srcv - base
                own = (loc >= 0) & (loc < HALF)
                key = jnp.where(own, 0, 1)
                pk = jnp.where(own, loc * 16384 + dstv,
                               jnp.full((16,), JUNK, jnp.int32))
                _, pk_sorted = plsc.sort_key_val(key, pk)
                cpk[pl.ds(fill, 16)] = pk_sorted
                fill = fill + jnp.sum(own.astype(jnp.int32))
                fired = fill >= CHUNK

                @pl.when(fired)
                def _():
                    fire(k)

                k = jnp.where(fired, k + 1, k)
                fill = jnp.where(fired, fill - CHUNK, fill)
        return fill, k

    fill, k = lax.fori_loop(0, NCH // BLKCH, block_body, (0, 0))
    for g in range(CHUNK // 16):
        cpk[pl.ds(fill + g * 16, 16)] = jnp.full((16,), JUNK, jnp.int32)
    fire(k)

    @pl.when(k % 2 == 0)
    def _():
        pltpu.make_async_copy(tbl.at[fs0], rows0, semg0).wait()
        pltpu.sync_copy(rows0, acc.at[fd0], add=True)

        @pl.when(k >= 1)
        def _():
            pltpu.make_async_copy(rows1, acc.at[fd1], sems1).wait()

    @pl.when(k % 2 == 1)
    def _():
        pltpu.make_async_copy(tbl.at[fs1], rows1, semg1).wait()
        pltpu.sync_copy(rows1, acc.at[fd1], add=True)
        pltpu.make_async_copy(rows0, acc.at[fd0], sems0).wait()

    plsc.subcore_barrier()
    pltpu.sync_copy(acc.at[pl.ds(s * RPT, RPT)],
                    out_hbm.at[c, pl.ds(s * RPT, RPT)])


_agg = functools.partial(
    pl.kernel,
    mesh=plsc.VectorSubcoreMesh(core_axis_name="c", subcore_axis_name="s"),
    compiler_params=_SC_PARAMS,
    out_type=jax.ShapeDtypeStruct((NC, ACC_ROWS, D), jnp.float32),
    scratch_types=[
        pltpu.VMEM((BLKCH, 2, CHUNK), jnp.int32),
        pltpu.VMEM((CAP,), jnp.int32),
        pltpu.VMEM((CHUNK,), jnp.int32),
        pltpu.VMEM((CHUNK,), jnp.int32),
        pltpu.VMEM((CHUNK,), jnp.int32),
        pltpu.VMEM((CHUNK,), jnp.int32),
        pltpu.VMEM((CHUNK, D), jnp.float32),
        pltpu.VMEM((CHUNK, D), jnp.float32),
        pltpu.VMEM_SHARED((ACC_ROWS, D), jnp.float32),
        pltpu.VMEM_SHARED((HALF, D), jnp.float32),
        pltpu.SemaphoreType.DMA,
        pltpu.SemaphoreType.DMA,
        pltpu.SemaphoreType.DMA,
        pltpu.SemaphoreType.DMA,
    ],
)(_agg_kernel)


CNT_B = 128
CNT_BLK = 16


def _cnt_kernel(idx_hbm, zeros_hbm, out_hbm, sdblk, fd0, fd1, ones, acc,
                sem0, sem1):
    c = lax.axis_index("c")
    s = lax.axis_index("s")
    pltpu.sync_copy(zeros_hbm, acc.at[pl.ds(s * RPT, RPT)])

    def ones_body(i, carry):
        ones[i] = jnp.ones((CNT_W,), jnp.float32)
        return carry

    lax.fori_loop(0, CNT_B, ones_body, 0)
    plsc.subcore_barrier()
    half_ch = NCH // NC
    fds = (fd0, fd1)
    sems = (sem0, sem1)

    def block_body(jb, carry):
        pltpu.sync_copy(
            idx_hbm.at[s, pl.ds(c * half_ch + jb * CNT_BLK, CNT_BLK)], sdblk)
        for f in range(4):
            fd = fds[f % 2]
            sem = sems[f % 2]
            if f < 2:
                @pl.when(jb > 0)
                def _():
                    pltpu.make_async_copy(ones, acc.at[fd], sem).wait()
            else:
                pltpu.make_async_copy(ones, acc.at[fd], sem).wait()
            for q in range(4):
                for g in range(CHUNK // 16):
                    fd[pl.ds(q * CHUNK + g * 16, 16)] = (
                        sdblk[f * 4 + q, 1, pl.ds(g * 16, 16)])
            pltpu.async_copy(ones, acc.at[fd], sem, add=True)
        return carry

    lax.fori_loop(0, half_ch // CNT_BLK, block_body, 0)
    pltpu.make_async_copy(ones, acc.at[fd0], sem0).wait()
    pltpu.make_async_copy(ones, acc.at[fd1], sem1).wait()
    plsc.subcore_barrier()
    pltpu.sync_copy(acc.at[pl.ds(s * RPT, RPT)],
                    out_hbm.at[c, pl.ds(s * RPT, RPT)])


_cnt = functools.partial(
    pl.kernel,
    mesh=plsc.VectorSubcoreMesh(core_axis_name="c", subcore_axis_name="s"),
    compiler_params=_SC_PARAMS,
    out_type=jax.ShapeDtypeStruct((NC, ACC_ROWS, CNT_W), jnp.float32),
    scratch_types=[
        pltpu.VMEM((CNT_BLK, 2, CHUNK), jnp.int32),
        pltpu.VMEM((CNT_B,), jnp.int32),
        pltpu.VMEM((CNT_B,), jnp.int32),
        pltpu.VMEM((CNT_B, CNT_W), jnp.float32),
        pltpu.VMEM_SHARED((ACC_ROWS, CNT_W), jnp.float32),
        pltpu.SemaphoreType.DMA,
        pltpu.SemaphoreType.DMA,
    ],
)(_cnt_kernel)


def _dot_t(a, w):
    return lax.dot_general(a, w, (((1,), (1,)), ((), ())),
                           preferred_element_type=jnp.float32)


def _prep1_body(x_ref, w1l_ref, w1r_ref, b1_ref, table_ref, xr_ref):
    xb = x_ref[...]
    table_ref[...] = _dot_t(xb, w1l_ref[...])
    xr_ref[...] = _dot_t(xb, w1r_ref[...]) + b1_ref[...]


def _prep2_body(p_ref, cnt_ref, xr1_ref, w2l_ref, w2r_ref, b2_ref,
                table_ref, xr_ref, inv_ref):
    agg = p_ref[0] + p_ref[1]
    cnt = (cnt_ref[0] + cnt_ref[1])[:, 0:1]
    inv = 1.0 / jnp.maximum(cnt, 1.0)
    h = agg * inv + xr1_ref[...]
    table_ref[...] = _dot_t(h, w2l_ref[...])
    xr_ref[...] = _dot_t(h, w2r_ref[...]) + b2_ref[...]
    inv_ref[...] = jnp.broadcast_to(inv, (BLK, D))


def _finish_body(q_ref, inv_ref, xr2_ref, out_ref):
    sm = q_ref[0] + q_ref[1]
    out_ref[...] = sm * inv_ref[...] + xr2_ref[...]


def _prep1(x, w1l, w1r, b1):
    return pl.pallas_call(
        _prep1_body,
        grid=(TBL_ROWS // BLK1,),
        in_specs=[
            pl.BlockSpec((BLK1, D), lambda i: (i, 0)),
            pl.BlockSpec((D, D), lambda i: (0, 0)),
            pl.BlockSpec((D, D), lambda i: (0, 0)),
            pl.BlockSpec((1, D), lambda i: (0, 0)),
        ],
        out_specs=[
            pl.BlockSpec((BLK1, D), lambda i: (i, 0)),
            pl.BlockSpec((BLK1, D), lambda i: (i, 0)),
        ],
        out_shape=[
            jax.ShapeDtypeStruct((TBL_ROWS, D), jnp.float32),
            jax.ShapeDtypeStruct((TBL_ROWS, D), jnp.float32),
        ],
    )(x, w1l, w1r, b1)


def _prep2(p, cntp, xr1, w2l, w2r, b2):
    return pl.pallas_call(
        _prep2_body,
        grid=(N_NODES // BLK,),
        in_specs=[
            pl.BlockSpec((NC, BLK, D), lambda i: (0, i, 0)),
            pl.BlockSpec((NC, BLK, CNT_W), lambda i: (0, i, 0)),
            pl.BlockSpec((BLK, D), lambda i: (i, 0)),
            pl.BlockSpec((D, D), lambda i: (0, 0)),
            pl.BlockSpec((D, D), lambda i: (0, 0)),
            pl.BlockSpec((1, D), lambda i: (0, 0)),
        ],
        out_specs=[
            pl.BlockSpec((BLK, D), lambda i: (i, 0)),
            pl.BlockSpec((BLK, D), lambda i: (i, 0)),
            pl.BlockSpec((BLK, D), lambda i: (i, 0)),
        ],
        out_shape=[
            jax.ShapeDtypeStruct((TBL_ROWS, D), jnp.float32),
            jax.ShapeDtypeStruct((N_NODES, D), jnp.float32),
            jax.ShapeDtypeStruct((N_NODES, D), jnp.float32),
        ],
    )(p, cntp, xr1, w2l, w2r, b2)


def _finish(q, inv, xr2):
    return pl.pallas_call(
        _finish_body,
        grid=(N_NODES // BLK,),
        in_specs=[
            pl.BlockSpec((NC, BLK, D), lambda i: (0, i, 0)),
            pl.BlockSpec((BLK, D), lambda i: (i, 0)),
            pl.BlockSpec((BLK, D), lambda i: (i, 0)),
        ],
        out_specs=pl.BlockSpec((BLK, D), lambda i: (i, 0)),
        out_shape=jax.ShapeDtypeStruct((N_NODES, D), jnp.float32),
    )(q, inv, xr2)


def kernel(x, edge_index, W1l, b1l, W1r, W2l, b2l, W2r):
    ei = edge_index.astype(jnp.int32)
    npad = E_PAD - N_EDGES
    src = jnp.concatenate([ei[0], jnp.zeros((npad,), jnp.int32)])
    dst = jnp.concatenate([ei[1], jnp.full((npad,), JUNK, jnp.int32)])
    idx = jnp.concatenate(
        [src.reshape(NS, NCH, 1, CHUNK), dst.reshape(NS, NCH, 1, CHUNK)],
        axis=2)
    xpad = jnp.concatenate(
        [x, jnp.zeros((TBL_ROWS - N_NODES, D), jnp.float32)])

    zeros_d = jnp.zeros((RPT, D), jnp.float32)
    zeros_c = jnp.zeros((RPT, CNT_W), jnp.float32)

    cntp = _cnt(idx, zeros_c)
    table1, xr1 = _prep1(xpad, W1l, W1r, b1l.reshape(1, D))
    p = _agg(idx, table1, zeros_d)
    table2, xr2, inv = _prep2(p, cntp, xr1, W2l, W2r, b2l.reshape(1, D))
    q = _agg(idx, table2, zeros_d)
    return _finish(q, inv, xr2)


# speedup vs baseline: 1.0026x; 1.0026x over previous
import functools

import jax
import jax.numpy as jnp
from jax import lax
from jax.experimental import pallas as pl
from jax.experimental.pallas import tpu as pltpu
from jax.experimental.pallas import tpu_sc as plsc

N_NODES = 10000
N_EDGES = 320000
D = 128

NC = 2
NS = 16
CHUNK = 32
NCH = 327680 // (NS * CHUNK)
BLKCH = 32
E_PAD = NS * NCH * CHUNK
TBL_ROWS = 10240
HALF = 5008
STG = HALF // NS
ACC_ROWS = 10016
RPT = ACC_ROWS // NS
JUNK = N_NODES
CAP = 128
CNT_W = 16
BLK1 = 512
BLK = 400

_SC_PARAMS = pltpu.CompilerParams(use_tc_tiling_on_sc=False,
                                  needs_layout_passes=False)


def _agg_kernel(idx_hbm, table_hbm, zeros_hbm, out_hbm,
                sdblk, cpk, fs0, fd0, fs1, fd1, rows0, rows1, acc, tbl,
                semg0, sems0, semg1, sems1):
    c = lax.axis_index("c")
    s = lax.axis_index("s")
    base = c * HALF
    pltpu.sync_copy(zeros_hbm, acc.at[pl.ds(s * RPT, RPT)])
    pltpu.sync_copy(table_hbm.at[pl.ds(base + s * STG, STG)],
                    tbl.at[pl.ds(s * STG, STG)])
    plsc.subcore_barrier()

    def fire_on(k, fs_a, fd_a, rows_a, semg_a, sems_a,
                fs_b, fd_b, rows_b, semg_b, sems_b):
        @pl.when(k >= 1)
        def _():
            pltpu.make_async_copy(tbl.at[fs_b], rows_b, semg_b).wait()
            pltpu.async_copy(rows_b, acc.at[fd_b], sems_b, add=True)

        @pl.when(k >= 2)
        def _():
            pltpu.make_async_copy(rows_a, acc.at[fd_a], sems_a).wait()

        for g in range(CHUNK // 16):
            v = cpk[pl.ds(g * 16, 16)]
            fs_a[pl.ds(g * 16, 16)] = lax.shift_right_logical(v, 14)
            fd_a[pl.ds(g * 16, 16)] = lax.bitwise_and(v, 16383)
        pltpu.async_copy(tbl.at[fs_a], rows_a, semg_a)
        cpk[pl.ds(0, 16)] = cpk[pl.ds(CHUNK, 16)]

    def fire(k):
        @pl.when(k % 2 == 0)
        def _():
            fire_on(k, fs0, fd0, rows0, semg0, sems0,
                    fs1, fd1, rows1, semg1, sems1)

        @pl.when(k % 2 == 1)
        def _():
            fire_on(k, fs1, fd1, rows1, semg1, sems1,
                    fs0, fd0, rows0, semg0, sems0)

    def block_body(jb, carry):
        fill, k = carry
        pltpu.sync_copy(idx_hbm.at[s, pl.ds(jb * BLKCH, BLKCH)], sdblk)
        for jj in range(BLKCH):
            for g in range(CHUNK // 16):
                srcv = sdblk[jj, 0, pl.ds(g * 16, 16)]
                dstv = sdblk[jj, 1, pl.ds(g * 16, 16)]
                loc = srcv - base
                own = (loc >= 0) & (loc < HALF)
                key = jnp.where(own, 0, 1)
                pk = jnp.where(own, loc * 16384 + dstv,
                               jnp.full((16,), JUNK, jnp.int32))
                _, pk_sorted = plsc.sort_key_val(key, pk)
                cpk[pl.ds(fill, 16)] = pk_sorted
                fill = fill + jnp.sum(own.astype(jnp.int32))
                fired = fill >= CHUNK

                @pl.when(fired)
                def _():
                    fire(k)

                k = jnp.where(fired, k + 1, k)
                fill = jnp.where(fired, fill - CHUNK, fill)
        return fill, k

    fill, k = lax.fori_loop(0, NCH // BLKCH, block_body, (0, 0))
    for g in range(CHUNK // 16):
        cpk[pl.ds(fill + g * 16, 16)] = jnp.full((16,), JUNK, jnp.int32)
    fire(k)

    @pl.when(k % 2 == 0)
    def _():
        pltpu.make_async_copy(tbl.at[fs0], rows0, semg0).wait()
        pltpu.sync_copy(rows0, acc.at[fd0], add=True)

        @pl.when(k >= 1)
        def _():
            pltpu.make_async_copy(rows1, acc.at[fd1], sems1).wait()

    @pl.when(k % 2 == 1)
    def _():
        pltpu.make_async_copy(tbl.at[fs1], rows1, semg1).wait()
        pltpu.sync_copy(rows1, acc.at[fd1], add=True)
        pltpu.make_async_copy(rows0, acc.at[fd0], sems0).wait()

    plsc.subcore_barrier()
    pltpu.sync_copy(acc.at[pl.ds(s * RPT, RPT)],
                    out_hbm.at[c, pl.ds(s * RPT, RPT)])


@functools.cache
def _agg():
    return functools.partial(
        pl.kernel,
        mesh=plsc.VectorSubcoreMesh(core_axis_name="c", subcore_axis_name="s"),
        compiler_params=_SC_PARAMS,
        out_type=jax.ShapeDtypeStruct((NC, ACC_ROWS, D), jnp.float32),
        scratch_types=[
            pltpu.VMEM((BLKCH, 2, CHUNK), jnp.int32),
            pltpu.VMEM((CAP,), jnp.int32),
            pltpu.VMEM((CHUNK,), jnp.int32),
            pltpu.VMEM((CHUNK,), jnp.int32),
            pltpu.VMEM((CHUNK,), jnp.int32),
            pltpu.VMEM((CHUNK,), jnp.int32),
            pltpu.VMEM((CHUNK, D), jnp.float32),
            pltpu.VMEM((CHUNK, D), jnp.float32),
            pltpu.VMEM_SHARED((ACC_ROWS, D), jnp.float32),
            pltpu.VMEM_SHARED((HALF, D), jnp.float32),
            pltpu.SemaphoreType.DMA,
            pltpu.SemaphoreType.DMA,
            pltpu.SemaphoreType.DMA,
            pltpu.SemaphoreType.DMA,
        ],
    )(_agg_kernel)


CNT_B = 128
CNT_BLK = 16


def _cnt_kernel(idx_hbm, zeros_hbm, out_hbm, sdblk, fd0, fd1, ones, acc,
                sem0, sem1):
    c = lax.axis_index("c")
    s = lax.axis_index("s")
    pltpu.sync_copy(zeros_hbm, acc.at[pl.ds(s * RPT, RPT)])

    def ones_body(i, carry):
        ones[i] = jnp.ones((CNT_W,), jnp.float32)
        return carry

    lax.fori_loop(0, CNT_B, ones_body, 0)
    plsc.subcore_barrier()
    half_ch = NCH // NC
    fds = (fd0, fd1)
    sems = (sem0, sem1)

    def block_body(jb, carry):
        pltpu.sync_copy(
            idx_hbm.at[s, pl.ds(c * half_ch + jb * CNT_BLK, CNT_BLK)], sdblk)
        for f in range(4):
            fd = fds[f % 2]
            sem = sems[f % 2]
            if f < 2:
                @pl.when(jb > 0)
                def _():
                    pltpu.make_async_copy(ones, acc.at[fd], sem).wait()
            else:
                pltpu.make_async_copy(ones, acc.at[fd], sem).wait()
            for q in range(4):
                for g in range(CHUNK // 16):
                    fd[pl.ds(q * CHUNK + g * 16, 16)] = (
                        sdblk[f * 4 + q, 1, pl.ds(g * 16, 16)])
            pltpu.async_copy(ones, acc.at[fd], sem, add=True)
        return carry

    lax.fori_loop(0, half_ch // CNT_BLK, block_body, 0)
    pltpu.make_async_copy(ones, acc.at[fd0], sem0).wait()
    pltpu.make_async_copy(ones, acc.at[fd1], sem1).wait()
    plsc.subcore_barrier()
    pltpu.sync_copy(acc.at[pl.ds(s * RPT, RPT)],
                    out_hbm.at[c, pl.ds(s * RPT, RPT)])


@functools.cache
def _cnt():
    return functools.partial(
        pl.kernel,
        mesh=plsc.VectorSubcoreMesh(core_axis_name="c", subcore_axis_name="s"),
        compiler_params=_SC_PARAMS,
        out_type=jax.ShapeDtypeStruct((NC, ACC_ROWS, CNT_W), jnp.float32),
        scratch_types=[
            pltpu.VMEM((CNT_BLK, 2, CHUNK), jnp.int32),
            pltpu.VMEM((CNT_B,), jnp.int32),
            pltpu.VMEM((CNT_B,), jnp.int32),
            pltpu.VMEM((CNT_B, CNT_W), jnp.float32),
            pltpu.VMEM_SHARED((ACC_ROWS, CNT_W), jnp.float32),
            pltpu.SemaphoreType.DMA,
            pltpu.SemaphoreType.DMA,
        ],
    )(_cnt_kernel)


def _dot_t(a, w):
    return lax.dot_general(a, w, (((1,), (1,)), ((), ())),
                           preferred_element_type=jnp.float32)


def _prep1_body(x_ref, w1l_ref, w1r_ref, b1_ref, table_ref, xr_ref):
    xb = x_ref[...]
    table_ref[...] = _dot_t(xb, w1l_ref[...])
    xr_ref[...] = _dot_t(xb, w1r_ref[...]) + b1_ref[...]


def _prep2_body(p_ref, cnt_ref, xr1_ref, w2l_ref, w2r_ref, b2_ref,
                table_ref, xr_ref, inv_ref):
    agg = p_ref[0] + p_ref[1]
    cnt = (cnt_ref[0] + cnt_ref[1])[:, 0:1]
    inv = 1.0 / jnp.maximum(cnt, 1.0)
    h = agg * inv + xr1_ref[...]
    table_ref[...] = _dot_t(h, w2l_ref[...])
    xr_ref[...] = _dot_t(h, w2r_ref[...]) + b2_ref[...]
    inv_ref[...] = jnp.broadcast_to(inv, (BLK, D))


def _finish_body(q_ref, inv_ref, xr2_ref, out_ref):
    sm = q_ref[0] + q_ref[1]
    out_ref[...] = sm * inv_ref[...] + xr2_ref[...]


def _prep1(x, w1l, w1r, b1):
    return pl.pallas_call(
        _prep1_body,
        grid=(TBL_ROWS // BLK1,),
        in_specs=[
            pl.BlockSpec((BLK1, D), lambda i: (i, 0)),
            pl.BlockSpec((D, D), lambda i: (0, 0)),
            pl.BlockSpec((D, D), lambda i: (0, 0)),
            pl.BlockSpec((1, D), lambda i: (0, 0)),
        ],
        out_specs=[
            pl.BlockSpec((BLK1, D), lambda i: (i, 0)),
            pl.BlockSpec((BLK1, D), lambda i: (i, 0)),
        ],
        out_shape=[
            jax.ShapeDtypeStruct((TBL_ROWS, D), jnp.float32),
            jax.ShapeDtypeStruct((TBL_ROWS, D), jnp.float32),
        ],
    )(x, w1l, w1r, b1)


def _prep2(p, cntp, xr1, w2l, w2r, b2):
    return pl.pallas_call(
        _prep2_body,
        grid=(N_NODES // BLK,),
        in_specs=[
            pl.BlockSpec((NC, BLK, D), lambda i: (0, i, 0)),
            pl.BlockSpec((NC, BLK, CNT_W), lambda i: (0, i, 0)),
            pl.BlockSpec((BLK, D), lambda i: (i, 0)),
            pl.BlockSpec((D, D), lambda i: (0, 0)),
            pl.BlockSpec((D, D), lambda i: (0, 0)),
            pl.BlockSpec((1, D), lambda i: (0, 0)),
        ],
        out_specs=[
            pl.BlockSpec((BLK, D), lambda i: (i, 0)),
            pl.BlockSpec((BLK, D), lambda i: (i, 0)),
            pl.BlockSpec((BLK, D), lambda i: (i, 0)),
        ],
        out_shape=[
            jax.ShapeDtypeStruct((TBL_ROWS, D), jnp.float32),
            jax.ShapeDtypeStruct((N_NODES, D), jnp.float32),
            jax.ShapeDtypeStruct((N_NODES, D), jnp.float32),
        ],
    )(p, cntp, xr1, w2l, w2r, b2)


def _finish(q, inv, xr2):
    return pl.pallas_call(
        _finish_body,
        grid=(N_NODES // BLK,),
        in_specs=[
            pl.BlockSpec((NC, BLK, D), lambda i: (0, i, 0)),
            pl.BlockSpec((BLK, D), lambda i: (i, 0)),
            pl.BlockSpec((BLK, D), lambda i: (i, 0)),
        ],
        out_specs=pl.BlockSpec((BLK, D), lambda i: (i, 0)),
        out_shape=jax.ShapeDtypeStruct((N_NODES, D), jnp.float32),
    )(q, inv, xr2)


def kernel(x, edge_index, W1l, b1l, W1r, W2l, b2l, W2r):
    ei = edge_index.astype(jnp.int32)
    npad = E_PAD - N_EDGES
    src = jnp.concatenate([ei[0], jnp.zeros((npad,), jnp.int32)])
    dst = jnp.concatenate([ei[1], jnp.full((npad,), JUNK, jnp.int32)])
    idx = jnp.concatenate(
        [src.reshape(NS, NCH, 1, CHUNK), dst.reshape(NS, NCH, 1, CHUNK)],
        axis=2)
    xpad = jnp.concatenate(
        [x, jnp.zeros((TBL_ROWS - N_NODES, D), jnp.float32)])

    zeros_d = jnp.zeros((RPT, D), jnp.float32)
    zeros_c = jnp.zeros((RPT, CNT_W), jnp.float32)

    agg = _agg()
    cntp = _cnt()(idx, zeros_c)
    table1, xr1 = _prep1(xpad, W1l, W1r, b1l.reshape(1, D))
    p = agg(idx, table1, zeros_d)
    table2, xr2, inv = _prep2(p, cntp, xr1, W2l, W2r, b2l.reshape(1, D))
    q = agg(idx, table2, zeros_d)
    return _finish(q, inv, xr2)
